# Initial kernel scaffold; baseline (speedup 1.0000x reference)
import jax
import jax.numpy as jnp
from jax.experimental import pallas as pl


def kernel(logits, edge_index, vertex):
    raise NotImplementedError("write your pallas kernel here")


# speedup vs baseline: 211.9037x; 211.9037x over previous
import functools

import jax
import jax.numpy as jnp
from jax import lax
from jax.experimental import pallas as pl
from jax.experimental.pallas import tpu as pltpu
from jax.experimental.pallas import tpu_sc as plsc

N_NODES = 100_000
N_EDGES = 6_400_000
N_PAD = 100_352
TRASH = N_NODES
NW = 32
BLK = 6_400
NBLK = N_EDGES // BLK
VPB = BLK // 16
INIT = N_PAD // 16

_mesh = plsc.VectorSubcoreMesh(core_axis_name="c", subcore_axis_name="s")


@functools.partial(
    pl.kernel,
    out_type=jax.ShapeDtypeStruct((2 * N_PAD,), jnp.float32),
    mesh=_mesh,
    compiler_params=pltpu.CompilerParams(needs_layout_passes=False),
    scratch_types=[
        pltpu.VMEM((BLK,), jnp.int32),
        pltpu.VMEM((BLK,), jnp.int32),
        pltpu.VMEM((INIT,), jnp.float32),
        pltpu.VMEM((16,), jnp.float32),
        pltpu.VMEM((16,), jnp.int32),
        pltpu.SemaphoreType.DMA,
    ],
)
def _sc_scan(edge_hbm, vtx_hbm, out_hbm, col_v, row_v, fill_v, zero_v,
             vtx_v, sem):
    c = lax.axis_index("c")
    s = lax.axis_index("s")
    wid = s * 2 + c

    pltpu.sync_copy(vtx_hbm, vtx_v)
    vv = vtx_v[...]

    zero_v[...] = jnp.zeros((16,), jnp.float32)
    minf = jnp.full((16,), -jnp.inf, jnp.float32)

    def fill_body(i, carry):
        fill_v[pl.ds(i * 16, 16)] = minf
        return carry

    lax.fori_loop(0, INIT // 16, fill_body, 0)
    pltpu.sync_copy(fill_v, out_hbm.at[pl.ds(c * N_PAD + s * INIT, INIT)])
    plsc.subcore_barrier()

    half = c * N_PAD

    def blk_body(j, carry):
        g = j * NW + wid

        @pl.when(g < NBLK)
        def _():
            off = g * BLK
            pltpu.sync_copy(edge_hbm.at[1, pl.ds(off, BLK)], col_v)

            def scan_body(i, acc):
                cv = col_v[pl.ds(i * 16, 16)]
                return acc | (cv == vv)

            acc = lax.fori_loop(0, VPB, scan_body,
                                jnp.zeros((16,), jnp.bool_))

            @pl.when(jnp.any(acc))
            def _():
                pltpu.sync_copy(edge_hbm.at[0, pl.ds(off, BLK)], row_v)

                def prec_body(i, carry):
                    cv = col_v[pl.ds(i * 16, 16)]

                    @pl.when(jnp.any(cv == vv))
                    def _():
                        rv = row_v[pl.ds(i * 16, 16)]
                        hit = (cv == vv) & (rv != vv)
                        idx = jnp.where(hit, rv + half, half + TRASH)
                        pltpu.async_copy(zero_v, out_hbm.at[idx], sem).wait()

                    return carry

                lax.fori_loop(0, VPB, prec_body, 0)

        return carry

    lax.fori_loop(0, (NBLK + NW - 1) // NW, blk_body, 0)


def _merge_body(vtx_ref, x_ref, o_ref):
    m = jnp.maximum(x_ref[0], x_ref[1])
    o_ref[...] = jnp.where(vtx_ref[0] == -1, jnp.float32(0.0), m)


_merge = pl.pallas_call(
    _merge_body,
    out_shape=jax.ShapeDtypeStruct((N_PAD // 128, 128), jnp.float32),
    in_specs=[
        pl.BlockSpec(memory_space=pltpu.SMEM),
        pl.BlockSpec(memory_space=pltpu.VMEM),
    ],
    out_specs=pl.BlockSpec(memory_space=pltpu.VMEM),
)


def kernel(logits, edge_index, vertex):
    del logits
    v = jnp.asarray(vertex, jnp.int32)
    vvec = jnp.full((16,), v, jnp.int32)
    halves = _sc_scan(edge_index, vvec)
    merged = _merge(v.reshape(1), halves.reshape(2, N_PAD // 128, 128))
    return merged.reshape(N_PAD)[:N_NODES].reshape(N_NODES, 1)


# speedup vs baseline: 270.9544x; 1.2787x over previous
import functools

import jax
import jax.numpy as jnp
from jax import lax
from jax.experimental import pallas as pl
from jax.experimental.pallas import tpu as pltpu
from jax.experimental.pallas import tpu_sc as plsc

N_NODES = 100_000
N_EDGES = 6_400_000
N_PAD = 100_352
TRASH = N_NODES
NW = 32
BLK = 6_400
NBLK = N_EDGES // BLK
VPB = BLK // 16
INIT = N_PAD // 16

_mesh = plsc.VectorSubcoreMesh(core_axis_name="c", subcore_axis_name="s")


@functools.partial(
    pl.kernel,
    out_type=jax.ShapeDtypeStruct((2 * N_PAD,), jnp.float32),
    mesh=_mesh,
    compiler_params=pltpu.CompilerParams(needs_layout_passes=False),
    scratch_types=[
        pltpu.VMEM((BLK,), jnp.int32),
        pltpu.VMEM((BLK,), jnp.int32),
        pltpu.VMEM((BLK,), jnp.int32),
        pltpu.VMEM((INIT,), jnp.float32),
        pltpu.VMEM((16,), jnp.float32),
        pltpu.VMEM((16,), jnp.int32),
        pltpu.SemaphoreType.DMA,
        pltpu.SemaphoreType.DMA,
        pltpu.SemaphoreType.DMA,
    ],
)
def _sc_scan(edge_hbm, vtx_hbm, out_hbm, cola_v, colb_v, row_v, fill_v,
             zero_v, vtx_v, sema, semb, semr):
    c = lax.axis_index("c")
    s = lax.axis_index("s")
    wid = s * 2 + c

    pltpu.sync_copy(vtx_hbm, vtx_v)
    vv = vtx_v[...]

    zero_v[...] = jnp.zeros((16,), jnp.float32)
    minf = jnp.full((16,), -jnp.inf, jnp.float32)

    @plsc.parallel_loop(0, INIT // 16, unroll=4)
    def _(i):
        fill_v[pl.ds(i * 16, 16)] = minf

    pltpu.sync_copy(fill_v, out_hbm.at[pl.ds(c * N_PAD + s * INIT, INIT)])
    plsc.subcore_barrier()

    half = c * N_PAD

    def start_fetch(g, buf, sem):
        return pltpu.async_copy(edge_hbm.at[1, pl.ds(g * BLK, BLK)], buf, sem)

    def scan_block(buf):
        false16 = jnp.zeros((16,), jnp.bool_)

        @plsc.parallel_loop(0, VPB, step=4, unroll=2,
                            carry=(false16, false16, false16, false16))
        def accs(i, acc):
            base = i * 16
            a0, a1, a2, a3 = acc
            return (
                a0 | (buf[pl.ds(base, 16)] == vv),
                a1 | (buf[pl.ds(base + 16, 16)] == vv),
                a2 | (buf[pl.ds(base + 32, 16)] == vv),
                a3 | (buf[pl.ds(base + 48, 16)] == vv),
            )

        a0, a1, a2, a3 = accs
        return jnp.any((a0 | a1) | (a2 | a3))

    def handle_block(g, buf):
        @pl.when(scan_block(buf))
        def _():
            pltpu.sync_copy(edge_hbm.at[0, pl.ds(g * BLK, BLK)], row_v)

            def prec_body(i, carry):
                cv = buf[pl.ds(i * 16, 16)]

                @pl.when(jnp.any(cv == vv))
                def _():
                    rv = row_v[pl.ds(i * 16, 16)]
                    hit = (cv == vv) & (rv != vv)
                    idx = jnp.where(hit, rv + half, half + TRASH)
                    pltpu.async_copy(zero_v, out_hbm.at[idx], semr).wait()

                return carry

            lax.fori_loop(0, VPB, prec_body, 0)

    start_fetch(wid, cola_v, sema)

    def blk_body(j2, carry):
        ja = 2 * j2
        ga = ja * NW + wid
        gb = ga + NW
        gc = gb + NW

        @pl.when(gb < NBLK)
        def _():
            start_fetch(gb, colb_v, semb)

        pltpu.make_async_copy(edge_hbm.at[1, pl.ds(ga * BLK, BLK)],
                              cola_v, sema).wait()
        handle_block(ga, cola_v)

        @pl.when(gc < NBLK)
        def _():
            start_fetch(gc, cola_v, sema)

        @pl.when(gb < NBLK)
        def _():
            pltpu.make_async_copy(edge_hbm.at[1, pl.ds(gb * BLK, BLK)],
                                  colb_v, semb).wait()
            handle_block(gb, colb_v)

        return carry

    lax.fori_loop(0, NBLK // (2 * NW) + 1, blk_body, 0)


def _merge_body(vtx_ref, x_ref, o_ref):
    m = jnp.maximum(x_ref[0], x_ref[1])
    o_ref[...] = jnp.where(vtx_ref[0] == -1, jnp.float32(0.0), m)


_merge = pl.pallas_call(
    _merge_body,
    out_shape=jax.ShapeDtypeStruct((N_PAD // 128, 128), jnp.float32),
    in_specs=[
        pl.BlockSpec(memory_space=pltpu.SMEM),
        pl.BlockSpec(memory_space=pltpu.VMEM),
    ],
    out_specs=pl.BlockSpec(memory_space=pltpu.VMEM),
)


def kernel(logits, edge_index, vertex):
    del logits
    v = jnp.asarray(vertex, jnp.int32)
    vvec = jnp.full((16,), v, jnp.int32)
    halves = _sc_scan(edge_index, vvec)
    merged = _merge(v.reshape(1), halves.reshape(2, N_PAD // 128, 128))
    return merged.reshape(N_PAD)[:N_NODES].reshape(N_NODES, 1)
